# baseline (speedup 1.0000x reference)
import functools

import jax
import jax.numpy as jnp
from jax import lax
from jax.experimental import pallas as pl
from jax.experimental.pallas import tpu as pltpu
from jax.experimental.pallas import tpu_sc as plsc

N_USERS = 5000
N_NODES = 10000
NC = 2
NS = 16
NW = NC * NS
LANES = 16
EDGE_B = 128


def _mesh():
    return plsc.VectorSubcoreMesh(core_axis_name="c", subcore_axis_name="s",
                                  num_cores=NC, num_subcores=NS)


def _sc_spmm(packed, vals_pk, Z, dv):
    TPW = packed.shape[1]
    D = Z.shape[1]
    CH = 80
    NCH = N_NODES // CH
    CPT = -(-NCH // NS)

    @functools.partial(
        pl.kernel,
        out_type=jax.ShapeDtypeStruct((NC, N_NODES, D), jnp.float32),
        mesh=_mesh(),
        scratch_types=[
            pltpu.VMEM((TPW, 2, EDGE_B), jnp.int32),
            pltpu.VMEM((TPW, EDGE_B), jnp.float32),
            pltpu.VMEM((2, EDGE_B), jnp.int32),
            pltpu.VMEM((2, EDGE_B, D), jnp.float32),
            pltpu.VMEM_SHARED((N_NODES, D), jnp.float32),
            pltpu.SemaphoreType.DMA,
            pltpu.SemaphoreType.DMA,
            pltpu.SemaphoreType.DMA,
            pltpu.SemaphoreType.DMA,
        ],
    )
    def k(packed_hbm, vals_hbm, z_hbm, out_hbm,
          ebuf, vbuf, ridx, zbuf, acc, sg0, sg1, ss0, ss1):
        c = lax.axis_index("c")
        s = lax.axis_index("s")
        wid = c * NS + s
        sg = (sg0, sg1)
        ss = (ss0, ss1)

        zero16 = jnp.zeros((LANES,), jnp.float32)

        def zrow(i, _):
            for j in range(D // LANES):
                zbuf[0, i, pl.ds(j * LANES, LANES)] = zero16
            return 0

        lax.fori_loop(0, CH, zrow, 0)
        for i in range(CPT):
            ch = s + i * NS

            @pl.when(ch < NCH)
            def _():
                pltpu.sync_copy(zbuf.at[0, pl.ds(0, CH)],
                                acc.at[pl.ds(ch * CH, CH)])

        pltpu.sync_copy(packed_hbm.at[wid], ebuf)
        pltpu.sync_copy(vals_hbm.at[wid], vbuf)
        plsc.subcore_barrier()

        def issue_gather(t, slot):
            pltpu.async_copy(z_hbm.at[ebuf.at[t, 1]], zbuf.at[slot], sg[slot])

        def wait_gather(slot):
            pltpu.make_async_copy(z_hbm.at[pl.ds(0, EDGE_B)],
                                  zbuf.at[slot], sg[slot]).wait()

        def wait_scatter(slot):
            pltpu.make_async_copy(z_hbm.at[pl.ds(0, EDGE_B)],
                                  zbuf.at[slot], ss[slot]).wait()

        def process(t, slot, first, last):
            o = 1 - slot
            if not last:
                if not first:
                    wait_scatter(o)
                issue_gather(t + 1, o)
            wait_gather(slot)
            for j in range(EDGE_B // LANES):
                sl = pl.ds(j * LANES, LANES)
                ridx[slot, sl] = ebuf[t, 0, sl]

            def edge_group(g, _):
                vv = vbuf[t, pl.ds(g * LANES, LANES)]
                for kk in range(LANES):
                    e = g * LANES + kk
                    for j in range(dv // LANES):
                        sl = pl.ds(j * LANES, LANES)
                        zbuf[slot, e, sl] = zbuf[slot, e, sl] * vv[kk]
                return 0

            lax.fori_loop(0, EDGE_B // LANES, edge_group, 0)
            pltpu.async_copy(zbuf.at[slot], acc.at[ridx.at[slot]],
                             ss[slot], add=True)

        issue_gather(0, 0)
        process(0, 0, True, False)
        process(1, 1, False, False)

        def pair(p, _):
            t0 = 2 * p
            process(t0, 0, False, False)
            process(t0 + 1, 1, False, False)
            return 0

        lax.fori_loop(1, TPW // 2 - 1, pair, 0)
        process(TPW - 2, 0, False, False)
        process(TPW - 1, 1, False, True)
        wait_scatter(0)
        wait_scatter(1)
        plsc.subcore_barrier()

        for i in range(CPT):
            ch = s + i * NS

            @pl.when(ch < NCH)
            def _():
                r0 = ch * CH
                pltpu.sync_copy(acc.at[pl.ds(r0, CH)],
                                zbuf.at[0, pl.ds(0, CH)])
                pltpu.sync_copy(zbuf.at[0, pl.ds(0, CH)],
                                out_hbm.at[c, pl.ds(r0, CH)])

    return k(packed, vals_pk, Z)


def _sc_gather(user_idx, item_idx2, feats0, feats1, p0, p1):
    B = user_idx.shape[0]
    RB = B // NW

    @functools.partial(
        pl.kernel,
        out_type=[
            jax.ShapeDtypeStruct((B, 256), jnp.float32),
            jax.ShapeDtypeStruct((B, 128), jnp.float32),
            jax.ShapeDtypeStruct((B, 64), jnp.float32),
            jax.ShapeDtypeStruct((B, 256), jnp.float32),
            jax.ShapeDtypeStruct((B, 128), jnp.float32),
            jax.ShapeDtypeStruct((B, 64), jnp.float32),
        ],
        mesh=_mesh(),
        scratch_types=[
            pltpu.VMEM((RB,), jnp.int32),
            pltpu.VMEM((RB, 256), jnp.float32),
            pltpu.VMEM((RB, 128), jnp.float32),
            pltpu.VMEM((RB, 128), jnp.float32),
            pltpu.VMEM((RB, 128), jnp.float32),
            pltpu.VMEM((RB, 64), jnp.float32),
            pltpu.SemaphoreType.DMA,
            pltpu.SemaphoreType.DMA,
        ],
    )
    def k(u_hbm, i_hbm, f0_hbm, f1_hbm, p0_hbm, p1_hbm,
          o0u, o1u, o2u, o0i, o1i, o2i,
          idx_v, g0, g1, ga, gb, gsum, semf, semp):
        c = lax.axis_index("c")
        s = lax.axis_index("s")
        wid = c * NS + s
        base = wid * RB

        for idx_hbm, o0, o1, o2 in ((u_hbm, o0u, o1u, o2u),
                                    (i_hbm, o0i, o1i, o2i)):
            pltpu.sync_copy(idx_hbm.at[pl.ds(base, RB)], idx_v)
            d0 = pltpu.async_copy(f0_hbm.at[idx_v], g0, semf)
            d1 = pltpu.async_copy(f1_hbm.at[idx_v], g1, semf)
            da = pltpu.async_copy(p0_hbm.at[idx_v], ga, semp)
            db = pltpu.async_copy(p1_hbm.at[idx_v], gb, semp)
            da.wait()
            db.wait()

            def addrow(i, _):
                for j in range(64 // LANES):
                    sl = pl.ds(j * LANES, LANES)
                    gsum[i, sl] = ga[i, sl] + gb[i, sl]
                return 0

            lax.fori_loop(0, RB, addrow, 0)
            pltpu.sync_copy(gsum, o2.at[pl.ds(base, RB)])
            d0.wait()
            d1.wait()
            pltpu.sync_copy(g0, o0.at[pl.ds(base, RB)])
            pltpu.sync_copy(g1, o1.at[pl.ds(base, RB)])

    return k(user_idx, item_idx2, feats0, feats1, p0, p1)


def _tc_layer(X, w1t, w2t, bsum, rb):
    n, din = X.shape
    dout = w1t.shape[1]

    def body(x_ref, w1_ref, w2_ref, b_ref, z_ref, ab_ref):
        x = x_ref[...]
        a = jnp.dot(x, w1_ref[...], preferred_element_type=jnp.float32)
        b = jnp.dot(x * x, w2_ref[...], preferred_element_type=jnp.float32)
        z_ref[...] = a + b
        ab_ref[...] = a + b_ref[...]

    return pl.pallas_call(
        body,
        grid=(n // rb,),
        in_specs=[
            pl.BlockSpec((rb, din), lambda i: (i, 0)),
            pl.BlockSpec((din, dout), lambda i: (0, 0)),
            pl.BlockSpec((din, dout), lambda i: (0, 0)),
            pl.BlockSpec((1, dout), lambda i: (0, 0)),
        ],
        out_specs=[
            pl.BlockSpec((rb, dout), lambda i: (i, 0)),
            pl.BlockSpec((rb, dout), lambda i: (i, 0)),
        ],
        out_shape=[
            jax.ShapeDtypeStruct((n, dout), jnp.float32),
            jax.ShapeDtypeStruct((n, dout), jnp.float32),
        ],
    )(X, w1t, w2t, bsum)


def _tc_layer2_in(p0, p1, a0b, w1t, w2t, rb):
    n, din = a0b.shape
    dout = w1t.shape[1]

    def body(p0_ref, p1_ref, ab_ref, w1_ref, w2_ref, f_ref, z_ref):
        f = p0_ref[...] + p1_ref[...] + ab_ref[...]
        f_ref[...] = f
        a = jnp.dot(f, w1_ref[...], preferred_element_type=jnp.float32)
        b = jnp.dot(f * f, w2_ref[...], preferred_element_type=jnp.float32)
        z_ref[...] = a + b

    return pl.pallas_call(
        body,
        grid=(n // rb,),
        in_specs=[
            pl.BlockSpec((rb, din), lambda i: (i, 0)),
            pl.BlockSpec((rb, din), lambda i: (i, 0)),
            pl.BlockSpec((rb, din), lambda i: (i, 0)),
            pl.BlockSpec((din, dout), lambda i: (0, 0)),
            pl.BlockSpec((din, dout), lambda i: (0, 0)),
        ],
        out_specs=[
            pl.BlockSpec((rb, din), lambda i: (i, 0)),
            pl.BlockSpec((rb, dout), lambda i: (i, 0)),
        ],
        out_shape=[
            jax.ShapeDtypeStruct((n, din), jnp.float32),
            jax.ShapeDtypeStruct((n, dout), jnp.float32),
        ],
    )(p0, p1, a0b, w1t, w2t)


def _tc_mlp(xs, w1s, w1t1, bsum1, t1b, t2t, t2b, t3, t3b, rb):
    n = xs[0].shape[0]
    dks = [x.shape[1] for x in xs]

    def body(*refs):
        x = refs[0:6]
        w = refs[6:12]
        w11_ref, bs1_ref = refs[12:14]
        b1_ref, w2_ref, b2_ref, w3_ref, b3_ref, o_ref = refs[14:]

        def dot(a, b):
            return jnp.dot(a, b, preferred_element_type=jnp.float32)

        h = b1_ref[...]
        for side in range(2):
            f0r, f1r, pr = x[3 * side:3 * side + 3]
            wa, wc, s2 = w[3 * side:3 * side + 3]
            f1 = f1r[...]
            f2 = pr[...] + dot(f1, w11_ref[...]) + bs1_ref[...]
            h = h + dot(f0r[...], wa[...]) + dot(f1, wc[...]) + dot(f2, s2[...])
        h = jax.nn.relu(h)
        h = jax.nn.relu(dot(h, w2_ref[...]) + b2_ref[...])
        o_ref[...] = jnp.sum(h * w3_ref[...], axis=1, keepdims=True) + b3_ref[...]

    return pl.pallas_call(
        body,
        grid=(n // rb,),
        in_specs=[pl.BlockSpec((rb, dk), lambda i: (i, 0)) for dk in dks]
        + [pl.BlockSpec((dk, 64), lambda i: (0, 0)) for dk in dks]
        + [
            pl.BlockSpec((128, 64), lambda i: (0, 0)),
            pl.BlockSpec((1, 64), lambda i: (0, 0)),
            pl.BlockSpec((1, 64), lambda i: (0, 0)),
            pl.BlockSpec((64, 32), lambda i: (0, 0)),
            pl.BlockSpec((1, 32), lambda i: (0, 0)),
            pl.BlockSpec((1, 32), lambda i: (0, 0)),
            pl.BlockSpec((1, 1), lambda i: (0, 0)),
        ],
        out_specs=pl.BlockSpec((rb, 1), lambda i: (i, 0)),
        out_shape=jax.ShapeDtypeStruct((n, 1), jnp.float32),
    )(*xs, *w1s, w1t1, bsum1, t1b, t2t, t2b, t3, t3b)


def kernel(userIdx, itemIdx, uEmbd, iEmbd, lap_row, lap_col, lap_val,
           ga0_W1, ga0_b1, ga0_W2, ga0_b2, ga1_W1, ga1_b1, ga1_W2, ga1_b2,
           t1_W, t1_b, t2_W, t2_b, t3_W, t3_b):
    feats0 = jnp.concatenate([uEmbd, iEmbd], axis=0)

    E = lap_row.shape[0]
    tpw = -(-E // (NW * EDGE_B))
    padn = NW * tpw * EDGE_B - E
    pad_ids = jnp.tile(jnp.arange(EDGE_B, dtype=lap_row.dtype), padn // EDGE_B)
    rows_p = jnp.concatenate([lap_row, pad_ids])
    cols_p = jnp.concatenate([lap_col, pad_ids])
    packed = jnp.stack([rows_p.reshape(-1, EDGE_B),
                        cols_p.reshape(-1, EDGE_B)],
                       axis=1).reshape(tpw, NW, 2, EDGE_B).transpose(1, 0, 2, 3)
    vals_pk = jnp.concatenate([lap_val, jnp.zeros((padn,), lap_val.dtype)]
                              ).reshape(tpw, NW, EDGE_B).transpose(1, 0, 2)

    z0, a0b = _tc_layer(feats0, ga0_W1.T, ga0_W2.T,
                        (ga0_b1 + ga0_b2)[None, :], 1000)
    part0 = _sc_spmm(packed, vals_pk, z0, 128)
    pad = jnp.zeros((128, 64), jnp.float32)
    w1t1 = jnp.concatenate([ga1_W1.T, pad], axis=1)
    w2t1 = jnp.concatenate([ga1_W2.T, pad], axis=1)
    feats1, z1 = _tc_layer2_in(part0[0], part0[1], a0b, w1t1, w2t1, 1000)
    part1 = _sc_spmm(packed, vals_pk, z1, 64)

    g0u, g1u, g2u, g0i, g1i, g2i = _sc_gather(
        userIdx, itemIdx + N_USERS, feats0, feats1, part1[0], part1[1])

    t1t = t1_W.T
    w1s = (t1t[0:256], t1t[256:384], t1t[384:448],
           t1t[448:704], t1t[704:832], t1t[832:896])
    bsum1 = ga1_b1 + ga1_b2
    out = _tc_mlp((g0u, g1u, g2u, g0i, g1i, g2i), w1s,
                  ga1_W1.T, bsum1[None, :],
                  t1_b[None, :], t2_W.T, t2_b[None, :],
                  t3_W, t3_b[None, :], 512)
    return out.reshape(-1)


# speedup vs baseline: 1.0247x; 1.0247x over previous
import functools

import jax
import jax.numpy as jnp
from jax import lax
from jax.experimental import pallas as pl
from jax.experimental.pallas import tpu as pltpu
from jax.experimental.pallas import tpu_sc as plsc

N_USERS = 5000
N_NODES = 10000
NC = 2
NS = 16
NW = NC * NS
LANES = 16
EDGE_B = 128


def _mesh():
    return plsc.VectorSubcoreMesh(core_axis_name="c", subcore_axis_name="s",
                                  num_cores=NC, num_subcores=NS)


def _sc_spmm(packed, vals_pk, Z, dv):
    TPW = packed.shape[1]
    D = Z.shape[1]
    CH = 80
    NCH = N_NODES // CH
    CPT = -(-NCH // NS)

    @functools.partial(
        pl.kernel,
        out_type=jax.ShapeDtypeStruct((NC, N_NODES, D), jnp.float32),
        mesh=_mesh(),
        scratch_types=[
            pltpu.VMEM((TPW, 2, EDGE_B), jnp.int32),
            pltpu.VMEM((TPW, EDGE_B), jnp.float32),
            pltpu.VMEM((2, EDGE_B), jnp.int32),
            pltpu.VMEM((2, EDGE_B, D), jnp.float32),
            pltpu.VMEM_SHARED((N_NODES, D), jnp.float32),
            pltpu.SemaphoreType.DMA,
            pltpu.SemaphoreType.DMA,
            pltpu.SemaphoreType.DMA,
            pltpu.SemaphoreType.DMA,
        ],
    )
    def k(packed_hbm, vals_hbm, z_hbm, out_hbm,
          ebuf, vbuf, ridx, zbuf, acc, sg0, sg1, ss0, ss1):
        c = lax.axis_index("c")
        s = lax.axis_index("s")
        wid = c * NS + s
        sg = (sg0, sg1)
        ss = (ss0, ss1)

        zero16 = jnp.zeros((LANES,), jnp.float32)

        def zrow(i, _):
            for j in range(D // LANES):
                zbuf[0, i, pl.ds(j * LANES, LANES)] = zero16
            return 0

        lax.fori_loop(0, CH, zrow, 0)
        for i in range(CPT):
            ch = s + i * NS

            @pl.when(ch < NCH)
            def _():
                pltpu.sync_copy(zbuf.at[0, pl.ds(0, CH)],
                                acc.at[pl.ds(ch * CH, CH)])

        pltpu.sync_copy(packed_hbm.at[wid], ebuf)
        pltpu.sync_copy(vals_hbm.at[wid], vbuf)
        plsc.subcore_barrier()

        def issue_gather(t, slot):
            pltpu.async_copy(z_hbm.at[ebuf.at[t, 1]], zbuf.at[slot], sg[slot])

        def wait_gather(slot):
            pltpu.make_async_copy(z_hbm.at[pl.ds(0, EDGE_B)],
                                  zbuf.at[slot], sg[slot]).wait()

        def wait_scatter(slot):
            pltpu.make_async_copy(z_hbm.at[pl.ds(0, EDGE_B)],
                                  zbuf.at[slot], ss[slot]).wait()

        def process(t, slot, first, last):
            o = 1 - slot
            if not last:
                if not first:
                    wait_scatter(o)
                issue_gather(t + 1, o)
            wait_gather(slot)
            for j in range(EDGE_B // LANES):
                sl = pl.ds(j * LANES, LANES)
                ridx[slot, sl] = ebuf[t, 0, sl]

            def edge_group(g, _):
                vv = vbuf[t, pl.ds(g * LANES, LANES)]
                for kk in range(LANES):
                    e = g * LANES + kk
                    for j in range(dv // LANES):
                        sl = pl.ds(j * LANES, LANES)
                        zbuf[slot, e, sl] = zbuf[slot, e, sl] * vv[kk]
                return 0

            lax.fori_loop(0, EDGE_B // LANES, edge_group, 0)
            pltpu.async_copy(zbuf.at[slot], acc.at[ridx.at[slot]],
                             ss[slot], add=True)

        issue_gather(0, 0)
        process(0, 0, True, False)
        process(1, 1, False, False)

        def pair(p, _):
            t0 = 2 * p
            process(t0, 0, False, False)
            process(t0 + 1, 1, False, False)
            return 0

        lax.fori_loop(1, TPW // 2 - 1, pair, 0)
        process(TPW - 2, 0, False, False)
        process(TPW - 1, 1, False, True)
        wait_scatter(0)
        wait_scatter(1)
        plsc.subcore_barrier()

        for i in range(CPT):
            ch = s + i * NS

            @pl.when(ch < NCH)
            def _():
                r0 = ch * CH
                pltpu.sync_copy(acc.at[pl.ds(r0, CH)],
                                zbuf.at[0, pl.ds(0, CH)])
                pltpu.sync_copy(zbuf.at[0, pl.ds(0, CH)],
                                out_hbm.at[c, pl.ds(r0, CH)])

    return k(packed, vals_pk, Z)


def _sc_gather(user_idx, item_idx2, feats0, feats1, p0, p1):
    B = user_idx.shape[0]
    RB = B // NW

    @functools.partial(
        pl.kernel,
        out_type=[
            jax.ShapeDtypeStruct((B, 256), jnp.float32),
            jax.ShapeDtypeStruct((B, 128), jnp.float32),
            jax.ShapeDtypeStruct((B, 64), jnp.float32),
            jax.ShapeDtypeStruct((B, 256), jnp.float32),
            jax.ShapeDtypeStruct((B, 128), jnp.float32),
            jax.ShapeDtypeStruct((B, 64), jnp.float32),
        ],
        mesh=_mesh(),
        scratch_types=[
            pltpu.VMEM((RB,), jnp.int32),
            pltpu.VMEM((RB, 256), jnp.float32),
            pltpu.VMEM((RB, 128), jnp.float32),
            pltpu.VMEM((RB, 128), jnp.float32),
            pltpu.VMEM((RB, 128), jnp.float32),
            pltpu.VMEM((RB, 64), jnp.float32),
            pltpu.SemaphoreType.DMA,
            pltpu.SemaphoreType.DMA,
        ],
    )
    def k(u_hbm, i_hbm, f0_hbm, f1_hbm, p0_hbm, p1_hbm,
          o0u, o1u, o2u, o0i, o1i, o2i,
          idx_v, g0, g1, ga, gb, gsum, semf, semp):
        c = lax.axis_index("c")
        s = lax.axis_index("s")
        wid = c * NS + s
        base = wid * RB

        for idx_hbm, o0, o1, o2 in ((u_hbm, o0u, o1u, o2u),
                                    (i_hbm, o0i, o1i, o2i)):
            pltpu.sync_copy(idx_hbm.at[pl.ds(base, RB)], idx_v)
            d0 = pltpu.async_copy(f0_hbm.at[idx_v], g0, semf)
            d1 = pltpu.async_copy(f1_hbm.at[idx_v], g1, semf)
            da = pltpu.async_copy(p0_hbm.at[idx_v], ga, semp)
            db = pltpu.async_copy(p1_hbm.at[idx_v], gb, semp)
            da.wait()
            db.wait()

            def addrow(i, _):
                for j in range(64 // LANES):
                    sl = pl.ds(j * LANES, LANES)
                    gsum[i, sl] = ga[i, sl] + gb[i, sl]
                return 0

            lax.fori_loop(0, RB, addrow, 0)
            pltpu.sync_copy(gsum, o2.at[pl.ds(base, RB)])
            d0.wait()
            d1.wait()
            pltpu.sync_copy(g0, o0.at[pl.ds(base, RB)])
            pltpu.sync_copy(g1, o1.at[pl.ds(base, RB)])

    return k(user_idx, item_idx2, feats0, feats1, p0, p1)


def _tc_layer(X, w1t, w2t, bsum, rb):
    n, din = X.shape
    dout = w1t.shape[1]

    def body(x_ref, w1_ref, w2_ref, b_ref, z_ref, ab_ref):
        x = x_ref[...]
        a = jnp.dot(x, w1_ref[...], preferred_element_type=jnp.float32)
        b = jnp.dot(x * x, w2_ref[...], preferred_element_type=jnp.float32)
        z_ref[...] = a + b
        ab_ref[...] = a + b_ref[...]

    return pl.pallas_call(
        body,
        grid=(n // rb,),
        in_specs=[
            pl.BlockSpec((rb, din), lambda i: (i, 0)),
            pl.BlockSpec((din, dout), lambda i: (0, 0)),
            pl.BlockSpec((din, dout), lambda i: (0, 0)),
            pl.BlockSpec((1, dout), lambda i: (0, 0)),
        ],
        out_specs=[
            pl.BlockSpec((rb, dout), lambda i: (i, 0)),
            pl.BlockSpec((rb, dout), lambda i: (i, 0)),
        ],
        out_shape=[
            jax.ShapeDtypeStruct((n, dout), jnp.float32),
            jax.ShapeDtypeStruct((n, dout), jnp.float32),
        ],
    )(X, w1t, w2t, bsum)


def _tc_layer2_in(p0, p1, a0b, w1t, w2t, rb):
    n, din = a0b.shape
    dout = w1t.shape[1]

    def body(p0_ref, p1_ref, ab_ref, w1_ref, w2_ref, f_ref, z_ref):
        f = p0_ref[...] + p1_ref[...] + ab_ref[...]
        f_ref[...] = f
        a = jnp.dot(f, w1_ref[...], preferred_element_type=jnp.float32)
        b = jnp.dot(f * f, w2_ref[...], preferred_element_type=jnp.float32)
        z_ref[...] = a + b

    return pl.pallas_call(
        body,
        grid=(n // rb,),
        in_specs=[
            pl.BlockSpec((rb, din), lambda i: (i, 0)),
            pl.BlockSpec((rb, din), lambda i: (i, 0)),
            pl.BlockSpec((rb, din), lambda i: (i, 0)),
            pl.BlockSpec((din, dout), lambda i: (0, 0)),
            pl.BlockSpec((din, dout), lambda i: (0, 0)),
        ],
        out_specs=[
            pl.BlockSpec((rb, din), lambda i: (i, 0)),
            pl.BlockSpec((rb, dout), lambda i: (i, 0)),
        ],
        out_shape=[
            jax.ShapeDtypeStruct((n, din), jnp.float32),
            jax.ShapeDtypeStruct((n, dout), jnp.float32),
        ],
    )(p0, p1, a0b, w1t, w2t)


def _tc_mlp(xs, w1s, w1t1, bsum1, t1b, t2t, t2b, t3, t3b, rb):
    n = xs[0].shape[0]
    dks = [x.shape[1] for x in xs]

    def body(*refs):
        x = refs[0:6]
        w = refs[6:12]
        w11_ref, bs1_ref = refs[12:14]
        b1_ref, w2_ref, b2_ref, w3_ref, b3_ref, o_ref = refs[14:]

        def dot(a, b):
            return jnp.dot(a, b, preferred_element_type=jnp.float32)

        h = b1_ref[...]
        for side in range(2):
            f0r, f1r, pr = x[3 * side:3 * side + 3]
            wa, wc, s2 = w[3 * side:3 * side + 3]
            f1 = f1r[...]
            f2 = pr[...] + dot(f1, w11_ref[...]) + bs1_ref[...]
            h = h + dot(f0r[...], wa[...]) + dot(f1, wc[...]) + dot(f2, s2[...])
        h = jax.nn.relu(h)
        h = jax.nn.relu(dot(h, w2_ref[...]) + b2_ref[...])
        o_ref[...] = jnp.sum(h * w3_ref[...], axis=1, keepdims=True) + b3_ref[...]

    return pl.pallas_call(
        body,
        grid=(n // rb,),
        in_specs=[pl.BlockSpec((rb, dk), lambda i: (i, 0)) for dk in dks]
        + [pl.BlockSpec((dk, 64), lambda i: (0, 0)) for dk in dks]
        + [
            pl.BlockSpec((128, 64), lambda i: (0, 0)),
            pl.BlockSpec((1, 64), lambda i: (0, 0)),
            pl.BlockSpec((1, 64), lambda i: (0, 0)),
            pl.BlockSpec((64, 32), lambda i: (0, 0)),
            pl.BlockSpec((1, 32), lambda i: (0, 0)),
            pl.BlockSpec((1, 32), lambda i: (0, 0)),
            pl.BlockSpec((1, 1), lambda i: (0, 0)),
        ],
        out_specs=pl.BlockSpec((rb, 1), lambda i: (i, 0)),
        out_shape=jax.ShapeDtypeStruct((n, 1), jnp.float32),
    )(*xs, *w1s, w1t1, bsum1, t1b, t2t, t2b, t3, t3b)


def kernel(userIdx, itemIdx, uEmbd, iEmbd, lap_row, lap_col, lap_val,
           ga0_W1, ga0_b1, ga0_W2, ga0_b2, ga1_W1, ga1_b1, ga1_W2, ga1_b2,
           t1_W, t1_b, t2_W, t2_b, t3_W, t3_b):
    feats0 = jnp.concatenate([uEmbd, iEmbd], axis=0)

    E = lap_row.shape[0]
    tpw = -(-E // (NW * EDGE_B))
    padn = NW * tpw * EDGE_B - E
    pad_ids = jnp.tile(jnp.arange(EDGE_B, dtype=lap_row.dtype), padn // EDGE_B)
    rows_p = jnp.concatenate([lap_row, pad_ids])
    cols_p = jnp.concatenate([lap_col, pad_ids])
    packed = jnp.stack([rows_p.reshape(-1, EDGE_B),
                        cols_p.reshape(-1, EDGE_B)],
                       axis=1).reshape(NW, tpw, 2, EDGE_B)
    vals_pk = jnp.concatenate([lap_val, jnp.zeros((padn,), lap_val.dtype)]
                              ).reshape(NW, tpw, EDGE_B)

    z0, a0b = _tc_layer(feats0, ga0_W1.T, ga0_W2.T,
                        (ga0_b1 + ga0_b2)[None, :], 1000)
    part0 = _sc_spmm(packed, vals_pk, z0, 128)
    pad = jnp.zeros((128, 64), jnp.float32)
    w1t1 = jnp.concatenate([ga1_W1.T, pad], axis=1)
    w2t1 = jnp.concatenate([ga1_W2.T, pad], axis=1)
    feats1, z1 = _tc_layer2_in(part0[0], part0[1], a0b, w1t1, w2t1, 1000)
    part1 = _sc_spmm(packed, vals_pk, z1, 64)

    g0u, g1u, g2u, g0i, g1i, g2i = _sc_gather(
        userIdx, itemIdx + N_USERS, feats0, feats1, part1[0], part1[1])

    t1t = t1_W.T
    w1s = (t1t[0:256], t1t[256:384], t1t[384:448],
           t1t[448:704], t1t[704:832], t1t[832:896])
    bsum1 = ga1_b1 + ga1_b2
    out = _tc_mlp((g0u, g1u, g2u, g0i, g1i, g2i), w1s,
                  ga1_W1.T, bsum1[None, :],
                  t1_b[None, :], t2_W.T, t2_b[None, :],
                  t3_W, t3_b[None, :], 512)
    return out.reshape(-1)
